# Initial kernel scaffold; baseline (speedup 1.0000x reference)
import jax
import jax.numpy as jnp
from jax.experimental import pallas as pl


def kernel(vertices, faces, face_edges, W_self_0, W_nb_0, b_0, W_self_1, W_nb_1, b_1, codebook):
    raise NotImplementedError("write your pallas kernel here")


# speedup vs baseline: 2.6777x; 2.6777x over previous
import functools

import jax
import jax.numpy as jnp
from jax import lax
from jax.experimental import pallas as pl
from jax.experimental.pallas import tpu as pltpu
from jax.experimental.pallas import tpu_sc as plsc

N_FACES = 10000
DIM = 192
N_EDGES = 320000
CODEBOOK_SIZE = 1024
NUM_QUANTIZERS = 2

NC = 2
NS = 16
NW = NC * NS

CHUNK = 128
E_PAD = 327680
EPT = E_PAD // NW
EPT_SC = E_PAD // NS
ROWS = 10240
RPT = ROWS // NS
PAD_DST = ROWS - 8
CNTW = 16
HALF = DIM // NC

BLK = 1000
GRID = N_FACES // BLK


def _sc_segment_sum(x_split, src_flat, dst_flat, zeros_chunk):
  mesh = plsc.VectorSubcoreMesh(core_axis_name="c", subcore_axis_name="s")

  @functools.partial(
      pl.kernel,
      out_type=jax.ShapeDtypeStruct((NC * ROWS, HALF), jnp.float32),
      mesh=mesh,
      scratch_types=[
          pltpu.VMEM((CHUNK,), jnp.int32),
          pltpu.VMEM((CHUNK,), jnp.int32),
          pltpu.VMEM((CHUNK, HALF), jnp.float32),
          pltpu.VMEM_SHARED((ROWS, HALF), jnp.float32),
          pltpu.SemaphoreType.DMA,
      ],
      compiler_params=pltpu.CompilerParams(use_tc_tiling_on_sc=False),
  )
  def agg_kernel(x_hbm, src_hbm, dst_hbm, zero_hbm, out_hbm,
                 src_v, dst_v, rows_v, acc_sh, sem):
    c = lax.axis_index("c")
    s = lax.axis_index("s")
    off = c * N_FACES

    pltpu.sync_copy(zero_hbm, rows_v)
    for z in range(RPT // CHUNK):
      pltpu.sync_copy(rows_v, acc_sh.at[pl.ds(s * RPT + z * CHUNK, CHUNK)])
    plsc.subcore_barrier()

    def body(i, carry):
      base = s * EPT_SC + i * CHUNK
      pltpu.sync_copy(src_hbm.at[pl.ds(base, CHUNK)], src_v)
      pltpu.sync_copy(dst_hbm.at[pl.ds(base, CHUNK)], dst_v)
      for k in range(CHUNK // 16):
        src_v[pl.ds(k * 16, 16)] = src_v[pl.ds(k * 16, 16)] + off
      pltpu.async_copy(x_hbm.at[src_v], rows_v, sem).wait()
      pltpu.sync_copy(rows_v, acc_sh.at[dst_v], add=True)
      return carry

    lax.fori_loop(0, EPT_SC // CHUNK, body, 0)
    plsc.subcore_barrier()

    for z in range(RPT // CHUNK):
      row = s * RPT + z * CHUNK
      pltpu.sync_copy(acc_sh.at[pl.ds(row, CHUNK)], rows_v)
      pltpu.sync_copy(rows_v, out_hbm.at[pl.ds(c * ROWS + row, CHUNK)])

  return agg_kernel(x_split, src_flat, dst_flat, zeros_chunk)


def _sc_counts(dst_flat, ones_chunk, zeros_chunk):
  mesh = plsc.VectorSubcoreMesh(core_axis_name="c", subcore_axis_name="s")

  @functools.partial(
      pl.kernel,
      out_type=jax.ShapeDtypeStruct((NC * ROWS, CNTW), jnp.float32),
      mesh=mesh,
      scratch_types=[
          pltpu.VMEM((CHUNK,), jnp.int32),
          pltpu.VMEM((CHUNK, CNTW), jnp.float32),
          pltpu.VMEM((CHUNK, CNTW), jnp.float32),
          pltpu.VMEM_SHARED((ROWS, CNTW), jnp.float32),
      ],
      compiler_params=pltpu.CompilerParams(use_tc_tiling_on_sc=False),
  )
  def cnt_kernel(dst_hbm, ones_hbm, zero_hbm, out_hbm,
                 dst_v, ones_v, buf_v, cnt_sh):
    c = lax.axis_index("c")
    s = lax.axis_index("s")
    t = c * NS + s

    pltpu.sync_copy(zero_hbm, buf_v)
    for z in range(RPT // CHUNK):
      pltpu.sync_copy(buf_v, cnt_sh.at[pl.ds(s * RPT + z * CHUNK, CHUNK)])
    pltpu.sync_copy(ones_hbm, ones_v)
    plsc.subcore_barrier()

    def body(i, carry):
      base = t * EPT + i * CHUNK
      pltpu.sync_copy(dst_hbm.at[pl.ds(base, CHUNK)], dst_v)
      pltpu.sync_copy(ones_v, cnt_sh.at[dst_v], add=True)
      return carry

    lax.fori_loop(0, EPT // CHUNK, body, 0)
    plsc.subcore_barrier()

    for z in range(RPT // CHUNK):
      row = s * RPT + z * CHUNK
      pltpu.sync_copy(cnt_sh.at[pl.ds(row, CHUNK)], buf_v)
      pltpu.sync_copy(buf_v, out_hbm.at[pl.ds(c * ROWS + row, CHUNK)])

  return cnt_kernel(dst_flat, ones_chunk, zeros_chunk)


def _dot_bf16(a, b):
  return jnp.dot(a.astype(jnp.bfloat16), b.astype(jnp.bfloat16),
                 preferred_element_type=jnp.float32)


def _combine_body(x_ref, sl_ref, sr_ref, c_ref, ws_ref, wn_ref, b_ref, o_ref):
  p = jnp.concatenate([sl_ref[0], sr_ref[0]], axis=-1)
  cnt = c_ref[0, :, :1] + c_ref[1, :, :1]
  cnt = jnp.maximum(cnt, 1.0)
  agg = p / cnt
  acc = _dot_bf16(x_ref[...], ws_ref[...])
  acc = acc + _dot_bf16(agg, wn_ref[...])
  o_ref[...] = acc + b_ref[...]


def _tc_combine(x, seg, counts, w_self, w_nb, b):
  return pl.pallas_call(
      _combine_body,
      grid=(GRID,),
      in_specs=[
          pl.BlockSpec((BLK, DIM), lambda i: (i, 0)),
          pl.BlockSpec((1, BLK, HALF), lambda i: (0, i, 0)),
          pl.BlockSpec((1, BLK, HALF), lambda i: (1, i, 0)),
          pl.BlockSpec((NC, BLK, CNTW), lambda i: (0, i, 0)),
          pl.BlockSpec((DIM, DIM), lambda i: (0, 0)),
          pl.BlockSpec((DIM, DIM), lambda i: (0, 0)),
          pl.BlockSpec((1, DIM), lambda i: (0, 0)),
      ],
      out_specs=pl.BlockSpec((BLK, DIM), lambda i: (i, 0)),
      out_shape=jax.ShapeDtypeStruct((N_FACES, DIM), jnp.float32),
  )(x, seg, seg, counts, w_self, w_nb, b.reshape(1, DIM))


def _rvq_body(x_ref, cb_ref, o_ref):
  cb = cb_ref[...]
  cb_sq = jnp.sum(cb * cb, axis=-1)
  x = x_ref[...]
  residual = x
  qtot = jnp.zeros_like(x)
  for _ in range(NUM_QUANTIZERS):
    cross = lax.dot_general(residual.astype(jnp.bfloat16),
                            cb.astype(jnp.bfloat16), (((1,), (1,)), ((), ())),
                            preferred_element_type=jnp.float32)
    dists = (jnp.sum(residual * residual, axis=-1, keepdims=True)
             - 2.0 * cross + cb_sq[None, :])
    idx = jnp.argmin(dists, axis=-1)
    onehot = (idx[:, None]
              == lax.broadcasted_iota(jnp.int32, (BLK, CODEBOOK_SIZE), 1))
    q = jnp.dot(onehot.astype(jnp.float32), cb,
                preferred_element_type=jnp.float32,
                precision=lax.Precision.HIGHEST)
    qtot = qtot + q
    residual = residual - q
  o_ref[...] = x + (qtot - x)


def _tc_rvq(x, codebook):
  return pl.pallas_call(
      _rvq_body,
      grid=(GRID,),
      in_specs=[
          pl.BlockSpec((BLK, DIM), lambda i: (i, 0)),
          pl.BlockSpec((CODEBOOK_SIZE, DIM), lambda i: (0, 0)),
      ],
      out_specs=pl.BlockSpec((BLK, DIM), lambda i: (i, 0)),
      out_shape=jax.ShapeDtypeStruct((N_FACES, DIM), jnp.float32),
  )(x, codebook)


def kernel(vertices, faces, face_edges, W_self_0, W_nb_0, b_0,
           W_self_1, W_nb_1, b_1, codebook):
  del vertices
  src = face_edges[0].astype(jnp.int32)
  dst = face_edges[1].astype(jnp.int32)
  pad = E_PAD - N_EDGES
  src_flat = jnp.concatenate([src, jnp.zeros((pad,), jnp.int32)])
  dst_flat = jnp.concatenate([dst, jnp.full((pad,), PAD_DST, jnp.int32)])

  zeros_chunk = jnp.zeros((CHUNK, HALF), jnp.float32)
  zeros_cnt = jnp.zeros((CHUNK, CNTW), jnp.float32)
  ones_cnt = jnp.ones((CHUNK, CNTW), jnp.float32)

  counts = _sc_counts(dst_flat, ones_cnt, zeros_cnt).reshape(NC, ROWS, CNTW)

  def split(x):
    return jnp.concatenate([x[:, :HALF], x[:, HALF:]], axis=0)

  seg0 = _sc_segment_sum(split(faces), src_flat, dst_flat,
                         zeros_chunk).reshape(NC, ROWS, HALF)
  x1 = _tc_combine(faces, seg0, counts, W_self_0, W_nb_0, b_0)

  seg1 = _sc_segment_sum(split(x1), src_flat, dst_flat,
                         zeros_chunk).reshape(NC, ROWS, HALF)
  x2 = _tc_combine(x1, seg1, counts, W_self_1, W_nb_1, b_1)

  return _tc_rvq(x2, codebook)


# speedup vs baseline: 3.6753x; 1.3725x over previous
import functools

import jax
import jax.numpy as jnp
from jax import lax
from jax.experimental import pallas as pl
from jax.experimental.pallas import tpu as pltpu
from jax.experimental.pallas import tpu_sc as plsc

N_FACES = 10000
DIM = 192
N_EDGES = 320000
CODEBOOK_SIZE = 1024
NUM_QUANTIZERS = 2

NC = 2
NS = 16
NW = NC * NS

CHUNK = 128
E_PAD = 327680
EPT = E_PAD // NW
EPT_SC = E_PAD // NS
ROWS = 10240
RPT = ROWS // NS
PAD_DST = ROWS - 8
CNTW = 16
HALF = DIM // NC

BLK = 1000
GRID = N_FACES // BLK


NSLOT = 5
NCH = EPT_SC // CHUNK


@functools.lru_cache(maxsize=1)
def _make_sc_segment_sum():
  mesh = plsc.VectorSubcoreMesh(core_axis_name="c", subcore_axis_name="s")

  @functools.partial(
      pl.kernel,
      out_type=jax.ShapeDtypeStruct((NC * ROWS, HALF), jnp.float32),
      mesh=mesh,
      scratch_types=(
          [pltpu.VMEM((CHUNK,), jnp.int32) for _ in range(2 * NSLOT)]
          + [pltpu.VMEM((CHUNK, HALF), jnp.float32) for _ in range(NSLOT)]
          + [pltpu.VMEM_SHARED((ROWS, HALF), jnp.float32)]
          + [pltpu.SemaphoreType.DMA for _ in range(3 * NSLOT)]),
      compiler_params=pltpu.CompilerParams(use_tc_tiling_on_sc=False),
  )
  def agg_kernel(x_hbm, src_hbm, dst_hbm, zero_hbm, out_hbm, *rest):
    sidx = rest[:NSLOT]
    didx = rest[NSLOT:2 * NSLOT]
    bufs = rest[2 * NSLOT:3 * NSLOT]
    acc_sh = rest[3 * NSLOT]
    isem = rest[3 * NSLOT + 1:3 * NSLOT + 1 + NSLOT]
    gsem = rest[3 * NSLOT + 1 + NSLOT:3 * NSLOT + 1 + 2 * NSLOT]
    ssem = rest[3 * NSLOT + 1 + 2 * NSLOT:]
    c = lax.axis_index("c")
    s = lax.axis_index("s")

    pltpu.sync_copy(zero_hbm, bufs[0])
    for z in range(RPT // CHUNK):
      pltpu.sync_copy(bufs[0], acc_sh.at[pl.ds(s * RPT + z * CHUNK, CHUNK)])
    plsc.subcore_barrier()

    def iissue(i, b):
      pltpu.async_copy(
          src_hbm.at[pl.ds(c * E_PAD + s * EPT_SC + i * CHUNK, CHUNK)],
          sidx[b], isem[b])
      pltpu.async_copy(
          dst_hbm.at[pl.ds(s * EPT_SC + i * CHUNK, CHUNK)], didx[b], isem[b])

    def iwait(i, b):
      pltpu.make_async_copy(
          src_hbm.at[pl.ds(c * E_PAD + s * EPT_SC + i * CHUNK, CHUNK)],
          sidx[b], isem[b]).wait()
      pltpu.make_async_copy(
          dst_hbm.at[pl.ds(s * EPT_SC + i * CHUNK, CHUNK)],
          didx[b], isem[b]).wait()

    def gissue(b):
      pltpu.async_copy(x_hbm.at[sidx[b]], bufs[b], gsem[b])

    def gwait(b):
      pltpu.make_async_copy(x_hbm.at[sidx[b]], bufs[b], gsem[b]).wait()

    def sissue(b):
      pltpu.async_copy(bufs[b], acc_sh.at[didx[b]], ssem[b], add=True)

    def swait(b):
      pltpu.make_async_copy(bufs[b], acc_sh.at[didx[b]], ssem[b]).wait()

    def step(i, iv):
      if 3 <= i < NCH + 3:
        swait((i - 3) % NSLOT)
      if i + 2 < NCH:
        iissue(iv + 2, (i + 2) % NSLOT)
      if i < NCH:
        iwait(iv, i % NSLOT)
        gissue(i % NSLOT)
      if 2 <= i < NCH + 2:
        gwait((i - 2) % NSLOT)
        sissue((i - 2) % NSLOT)

    iissue(0, 0)
    iissue(1, 1)
    for i in range(NSLOT):
      step(i, i)

    def steady(o, carry):
      for b in range(NSLOT):
        i = NSLOT + b
        step(i, o * NSLOT + b)
      return carry

    lax.fori_loop(1, NCH // NSLOT - 1, steady, 0)

    for i in range(NCH - NSLOT, NCH + 3):
      step(i, i)
    plsc.subcore_barrier()

    for z in range(RPT // CHUNK):
      row = s * RPT + z * CHUNK
      pltpu.sync_copy(acc_sh.at[pl.ds(row, CHUNK)], bufs[0])
      pltpu.sync_copy(bufs[0], out_hbm.at[pl.ds(c * ROWS + row, CHUNK)])

  return agg_kernel


def _sc_segment_sum(x_split, src_flat2, dst_flat, zeros_chunk):
  return _make_sc_segment_sum()(x_split, src_flat2, dst_flat, zeros_chunk)


def _sc_counts(dst_flat, ones_chunk, zeros_chunk):
  mesh = plsc.VectorSubcoreMesh(core_axis_name="c", subcore_axis_name="s")

  @functools.partial(
      pl.kernel,
      out_type=jax.ShapeDtypeStruct((NC * ROWS, CNTW), jnp.float32),
      mesh=mesh,
      scratch_types=[
          pltpu.VMEM((CHUNK,), jnp.int32),
          pltpu.VMEM((CHUNK, CNTW), jnp.float32),
          pltpu.VMEM((CHUNK, CNTW), jnp.float32),
          pltpu.VMEM_SHARED((ROWS, CNTW), jnp.float32),
      ],
      compiler_params=pltpu.CompilerParams(use_tc_tiling_on_sc=False),
  )
  def cnt_kernel(dst_hbm, ones_hbm, zero_hbm, out_hbm,
                 dst_v, ones_v, buf_v, cnt_sh):
    c = lax.axis_index("c")
    s = lax.axis_index("s")
    t = c * NS + s

    pltpu.sync_copy(zero_hbm, buf_v)
    for z in range(RPT // CHUNK):
      pltpu.sync_copy(buf_v, cnt_sh.at[pl.ds(s * RPT + z * CHUNK, CHUNK)])
    pltpu.sync_copy(ones_hbm, ones_v)
    plsc.subcore_barrier()

    def body(i, carry):
      base = t * EPT + i * CHUNK
      pltpu.sync_copy(dst_hbm.at[pl.ds(base, CHUNK)], dst_v)
      pltpu.sync_copy(ones_v, cnt_sh.at[dst_v], add=True)
      return carry

    lax.fori_loop(0, EPT // CHUNK, body, 0)
    plsc.subcore_barrier()

    for z in range(RPT // CHUNK):
      row = s * RPT + z * CHUNK
      pltpu.sync_copy(cnt_sh.at[pl.ds(row, CHUNK)], buf_v)
      pltpu.sync_copy(buf_v, out_hbm.at[pl.ds(c * ROWS + row, CHUNK)])

  return cnt_kernel(dst_flat, ones_chunk, zeros_chunk)


def _dot_bf16(a, b):
  return jnp.dot(a.astype(jnp.bfloat16), b.astype(jnp.bfloat16),
                 preferred_element_type=jnp.float32)


def _combine_body(x_ref, sl_ref, sr_ref, c_ref, ws_ref, wn_ref, b_ref, o_ref):
  p = jnp.concatenate([sl_ref[0], sr_ref[0]], axis=-1)
  cnt = c_ref[0, :, :1] + c_ref[1, :, :1]
  cnt = jnp.maximum(cnt, 1.0)
  agg = p / cnt
  acc = _dot_bf16(x_ref[...], ws_ref[...])
  acc = acc + _dot_bf16(agg, wn_ref[...])
  o_ref[...] = acc + b_ref[...]


def _tc_combine(x, seg, counts, w_self, w_nb, b):
  return pl.pallas_call(
      _combine_body,
      grid=(GRID,),
      in_specs=[
          pl.BlockSpec((BLK, DIM), lambda i: (i, 0)),
          pl.BlockSpec((1, BLK, HALF), lambda i: (0, i, 0)),
          pl.BlockSpec((1, BLK, HALF), lambda i: (1, i, 0)),
          pl.BlockSpec((NC, BLK, CNTW), lambda i: (0, i, 0)),
          pl.BlockSpec((DIM, DIM), lambda i: (0, 0)),
          pl.BlockSpec((DIM, DIM), lambda i: (0, 0)),
          pl.BlockSpec((1, DIM), lambda i: (0, 0)),
      ],
      out_specs=pl.BlockSpec((BLK, DIM), lambda i: (i, 0)),
      out_shape=jax.ShapeDtypeStruct((N_FACES, DIM), jnp.float32),
  )(x, seg, seg, counts, w_self, w_nb, b.reshape(1, DIM))


def _rvq_body(x_ref, cb_ref, o_ref):
  cb = cb_ref[...]
  cb_sq = jnp.sum(cb * cb, axis=-1)
  x = x_ref[...]
  residual = x
  qtot = jnp.zeros_like(x)
  for _ in range(NUM_QUANTIZERS):
    cross = lax.dot_general(residual.astype(jnp.bfloat16),
                            cb.astype(jnp.bfloat16), (((1,), (1,)), ((), ())),
                            preferred_element_type=jnp.float32)
    dists = (jnp.sum(residual * residual, axis=-1, keepdims=True)
             - 2.0 * cross + cb_sq[None, :])
    idx = jnp.argmin(dists, axis=-1)
    onehot = (idx[:, None]
              == lax.broadcasted_iota(jnp.int32, (BLK, CODEBOOK_SIZE), 1))
    q = jnp.dot(onehot.astype(jnp.float32), cb,
                preferred_element_type=jnp.float32,
                precision=lax.Precision.HIGHEST)
    qtot = qtot + q
    residual = residual - q
  o_ref[...] = x + (qtot - x)


def _tc_rvq(x, codebook):
  return pl.pallas_call(
      _rvq_body,
      grid=(GRID,),
      in_specs=[
          pl.BlockSpec((BLK, DIM), lambda i: (i, 0)),
          pl.BlockSpec((CODEBOOK_SIZE, DIM), lambda i: (0, 0)),
      ],
      out_specs=pl.BlockSpec((BLK, DIM), lambda i: (i, 0)),
      out_shape=jax.ShapeDtypeStruct((N_FACES, DIM), jnp.float32),
  )(x, codebook)


def kernel(vertices, faces, face_edges, W_self_0, W_nb_0, b_0,
           W_self_1, W_nb_1, b_1, codebook):
  del vertices
  src = face_edges[0].astype(jnp.int32)
  dst = face_edges[1].astype(jnp.int32)
  pad = E_PAD - N_EDGES
  src_flat = jnp.concatenate([src, jnp.zeros((pad,), jnp.int32)])
  dst_flat = jnp.concatenate([dst, jnp.full((pad,), PAD_DST, jnp.int32)])
  src_flat2 = jnp.concatenate([src_flat, src_flat + N_FACES])

  zeros_chunk = jnp.zeros((CHUNK, HALF), jnp.float32)
  zeros_cnt = jnp.zeros((CHUNK, CNTW), jnp.float32)
  ones_cnt = jnp.ones((CHUNK, CNTW), jnp.float32)

  counts = _sc_counts(dst_flat, ones_cnt, zeros_cnt).reshape(NC, ROWS, CNTW)

  def split(x):
    return jnp.concatenate([x[:, :HALF], x[:, HALF:]], axis=0)

  seg0 = _sc_segment_sum(split(faces), src_flat2, dst_flat,
                         zeros_chunk).reshape(NC, ROWS, HALF)
  x1 = _tc_combine(faces, seg0, counts, W_self_0, W_nb_0, b_0)

  seg1 = _sc_segment_sum(split(x1), src_flat2, dst_flat,
                         zeros_chunk).reshape(NC, ROWS, HALF)
  x2 = _tc_combine(x1, seg1, counts, W_self_1, W_nb_1, b_1)

  return _tc_rvq(x2, codebook)
